# baseline (speedup 1.0000x reference)
import jax
import jax.numpy as jnp
from jax import lax
from jax.experimental import pallas as pl
from jax.experimental.pallas import tpu as pltpu
from jax.experimental.pallas import tpu_sc as plsc

D = 128
DEG = 16
NPAIR = 16
CH = 96
CH0 = 15
CH1 = 5
PAIR_N = (CH0 + CH1) * CH
BP = NPAIR * PAIR_N
STAGE = CH0 * CH
BPS = BP + STAGE - CH1 * CH


def _sc_gather_body(table, nodes, neigh_t, self_out, sum_out,
                    nidx, gidx, selfb, acc, ra, rb, sem_s, sem_a, sem_b):
    c = lax.axis_index("c")
    s = lax.axis_index("s")
    base = pl.multiple_of(s * PAIR_N + c * STAGE, 16)
    nchunks = jnp.where(c == 0, CH0, CH1)

    @pl.when(nchunks > 0)
    def _stage():
        pltpu.sync_copy(nodes.at[pl.ds(base, STAGE)], nidx)

        def stage_body(j, cc):
            pltpu.sync_copy(
                neigh_t.at[pl.ds(pl.multiple_of(j * BPS + base, 8), STAGE)],
                gidx.at[pl.ds(pl.multiple_of(j * STAGE, 8), STAGE)])
            return cc
        lax.fori_loop(0, DEG, stage_body, 0)

    def chunk_body(i, carry):
      @pl.when(i < nchunks)
      def _run():
        off = pl.multiple_of(i * CH, 16)
        cbase = base + off
        cp_self = pltpu.async_copy(table.at[nidx.at[pl.ds(off, CH)]],
                                   selfb, sem_s)
        cp0 = pltpu.async_copy(table.at[gidx.at[pl.ds(off, CH)]],
                               ra, sem_a)
        cp1 = pltpu.async_copy(table.at[gidx.at[pl.ds(STAGE + off, CH)]],
                               rb, sem_b)
        cps = {0: cp0, 1: cp1}
        for j in range(DEG):
            buf = ra if (j % 2 == 0) else rb
            sem = sem_a if (j % 2 == 0) else sem_b
            cps.pop(j).wait()

            def acc_body(it, cc, buf=buf, first=(j == 0)):
                rbase = it * 8
                for rr in range(8):
                    r = rbase + rr
                    for d in range(D // 16):
                        sl = pl.ds(d * 16, 16)
                        if first:
                            acc[r, sl] = buf[r, sl]
                        else:
                            plsc.addupdate(acc.at[r, sl], buf[r, sl])
                return cc
            lax.fori_loop(0, CH // 8, acc_body, 0)

            if j + 2 < DEG:
                cps[j + 2] = pltpu.async_copy(
                    table.at[gidx.at[pl.ds((j + 2) * STAGE + off, CH)]],
                    buf, sem)

        cp_self.wait()
        pltpu.sync_copy(selfb, self_out.at[pl.ds(cbase, CH)])
        pltpu.sync_copy(acc, sum_out.at[pl.ds(cbase, CH)])
      return carry

    lax.fori_loop(0, CH0, chunk_body, 0)


def _sc_gather(table, nodes_p, neigh_t):
    run = pl.kernel(
        _sc_gather_body,
        mesh=plsc.VectorSubcoreMesh(core_axis_name="c", subcore_axis_name="s"),
        out_type=(jax.ShapeDtypeStruct((BP, D), jnp.float32),
                  jax.ShapeDtypeStruct((BP, D), jnp.float32)),
        scratch_types=[
            pltpu.VMEM((STAGE,), jnp.int32),
            pltpu.VMEM((DEG * STAGE,), jnp.int32),
            pltpu.VMEM((CH, D), jnp.float32),
            pltpu.VMEM((CH, D), jnp.float32),
            pltpu.VMEM((CH, D), jnp.float32),
            pltpu.VMEM((CH, D), jnp.float32),
            pltpu.SemaphoreType.DMA,
            pltpu.SemaphoreType.DMA,
            pltpu.SemaphoreType.DMA,
        ],
    )
    return run(table, nodes_p, neigh_t)


def _mm_body(a1_ref, a2_ref, w1_ref, w2_ref, b_ref, o_ref):
    acc = jnp.dot(a1_ref[...], w1_ref[...], preferred_element_type=jnp.float32)
    acc = acc + jnp.dot(a2_ref[...], w2_ref[...],
                        preferred_element_type=jnp.float32)
    o_ref[...] = jnp.maximum(acc + b_ref[...], 0.0)


def _tc_linear(self_p, sum_p, w1, w2, b2d, batch):
    blk = 1200
    return pl.pallas_call(
        _mm_body,
        grid=(batch // blk,),
        in_specs=[
            pl.BlockSpec((blk, D), lambda i: (i, 0)),
            pl.BlockSpec((blk, D), lambda i: (i, 0)),
            pl.BlockSpec((D, D), lambda i: (0, 0)),
            pl.BlockSpec((D, D), lambda i: (0, 0)),
            pl.BlockSpec((1, D), lambda i: (0, 0)),
        ],
        out_specs=pl.BlockSpec((blk, D), lambda i: (i, 0)),
        out_shape=jax.ShapeDtypeStruct((batch, D), jnp.float32),
    )(self_p, sum_p, w1, w2, b2d)


def kernel(nodes, neigh_idx, feat_table, W, b):
    batch = nodes.shape[0]
    pad = BPS - batch
    nodes_p = jnp.concatenate([nodes, jnp.zeros((pad,), jnp.int32)])
    neigh_p = jnp.concatenate(
        [neigh_idx, jnp.zeros((pad, DEG), jnp.int32)], axis=0)
    neigh_t = neigh_p.T.reshape(-1)
    w1 = W[:D]
    w2 = W[D:] * (1.0 / DEG)
    self_p, sum_p = _sc_gather(feat_table, nodes_p, neigh_t)
    return _tc_linear(self_p, sum_p, w1, w2, b.reshape(1, D), batch)
